# baseline (speedup 1.0000x reference)
import functools

import jax
import jax.numpy as jnp
from jax import lax
from jax.experimental import pallas as pl
from jax.experimental.pallas import tpu as pltpu
from jax.experimental.pallas import tpu_sc as plsc

NC = 2
NS = 16
NW = NC * NS
L = 16

IDXW = 128
CHUNK = 640
NSUB = CHUNK // IDXW

BR = 1600


def _repack_kernel(q_ref, out_ref):
    qb = q_ref[...].astype(jnp.int32) & 255
    shift = (jax.lax.broadcasted_iota(jnp.int32, (1, 64), 1) & 3) << 3
    t = qb << shift
    words = t.reshape(BR, 16, 4).sum(axis=2)
    out_ref[...] = words.reshape(BR // 8, 128)


def _repack(q_weight):
    V, D = q_weight.shape
    return pl.pallas_call(
        _repack_kernel,
        out_shape=jax.ShapeDtypeStruct((V // 8, 128), jnp.int32),
        grid=(V // BR,),
        in_specs=[pl.BlockSpec((BR, D), lambda i: (i, 0))],
        out_specs=pl.BlockSpec((BR // 8, 128), lambda i: (i, 0)),
    )(q_weight)


def _dequant_chunk(rows_v, amax_v, out_v, iota16):
    inv127 = jnp.float32(1.0 / 127.0)

    def grp(g, _):
        r0 = g * L
        scale16 = amax_v[pl.ds(r0, L)] * inv127
        for rr in range(L):
            r = r0 + rr
            words = rows_v[r]
            scale_b = jnp.broadcast_to(scale16[rr], (L,))
            obase = r * 64
            for j in range(4):
                if j < 3:
                    b = (words << (24 - 8 * j)) >> 24
                else:
                    b = words >> 24
                f = b.astype(jnp.float32) * scale_b
                plsc.store_scatter(out_v, [obase + 4 * iota16 + j], f)
        return 0

    lax.fori_loop(0, CHUNK // L, grp, 0)


def _make_sc_kernel(N, V):
    per_w = N // NW
    nchunks = per_w // CHUNK
    mesh = plsc.VectorSubcoreMesh(
        core_axis_name="c", subcore_axis_name="s", num_cores=NC, num_subcores=NS)

    @functools.partial(
        pl.kernel,
        out_type=jax.ShapeDtypeStruct((N * 64,), jnp.float32),
        mesh=mesh,
        compiler_params=pltpu.CompilerParams(
            use_tc_tiling_on_sc=False, needs_layout_passes=False),
        scratch_types=[
            pltpu.VMEM((CHUNK,), jnp.int32),
            pltpu.VMEM((CHUNK, 16), jnp.int32),
            pltpu.VMEM((CHUNK,), jnp.float32),
            pltpu.VMEM((CHUNK * 64,), jnp.float32),
            pltpu.SemaphoreType.DMA,
            pltpu.SemaphoreType.DMA,
        ],
    )
    def k(idx_hbm, wtab_hbm, amax_hbm, out_hbm,
          idx_v, rows_v, amax_v, out_v, sem_r, sem_a):
        wid = lax.axis_index("s") * NC + lax.axis_index("c")
        iota16 = lax.iota(jnp.int32, L)

        def body(ci, _):
            base = wid * per_w + ci * CHUNK
            pltpu.sync_copy(idx_hbm.at[pl.ds(base, CHUNK)], idx_v)
            for s in range(NSUB):
                pltpu.async_copy(
                    wtab_hbm.at[idx_v.at[pl.ds(s * IDXW, IDXW)]],
                    rows_v.at[pl.ds(s * IDXW, IDXW)], sem_r)
                pltpu.async_copy(
                    amax_hbm.at[idx_v.at[pl.ds(s * IDXW, IDXW)]],
                    amax_v.at[pl.ds(s * IDXW, IDXW)], sem_a)
            for s in range(NSUB):
                pltpu.make_async_copy(
                    wtab_hbm.at[idx_v.at[pl.ds(s * IDXW, IDXW)]],
                    rows_v.at[pl.ds(s * IDXW, IDXW)], sem_r).wait()
                pltpu.make_async_copy(
                    amax_hbm.at[idx_v.at[pl.ds(s * IDXW, IDXW)]],
                    amax_v.at[pl.ds(s * IDXW, IDXW)], sem_a).wait()
            _dequant_chunk(rows_v, amax_v, out_v, iota16)
            pltpu.sync_copy(out_v, out_hbm.at[pl.ds(base * 64, CHUNK * 64)])
            return 0

        lax.fori_loop(0, nchunks, body, 0)

    return k


def kernel(x, q_weight, absmax):
    B, S = x.shape
    V, D = q_weight.shape
    N = B * S
    idx = x.reshape(N).astype(jnp.int32)
    wtab = _repack(q_weight).reshape(V, 16)
    out = _make_sc_kernel(N, V)(idx, wtab, absmax)
    return out.reshape(B, S, D)


# speedup vs baseline: 8.4964x; 8.4964x over previous
import functools

import jax
import jax.numpy as jnp
from jax import lax
from jax.experimental import pallas as pl
from jax.experimental.pallas import tpu as pltpu
from jax.experimental.pallas import tpu_sc as plsc

NC = 2
NS = 16
NW = NC * NS
L = 16

IDXW = 128
CHUNK = 640
NSUB = CHUNK // IDXW

BRT = 1000


def _repack_kernel(q_ref, out_ref):
    qf = (q_ref[...].astype(jnp.int32) & 255).astype(jnp.float32)
    c = jax.lax.broadcasted_iota(jnp.int32, (512, 128), 0)
    m4 = jax.lax.broadcasted_iota(jnp.int32, (512, 128), 1) * 4
    wlow = ((c == m4) + (c == m4 + 1) * 256).astype(jnp.float32)
    whigh = ((c == m4 + 2) + (c == m4 + 3) * 256).astype(jnp.float32)
    low = jnp.dot(qf, wlow, preferred_element_type=jnp.float32)
    high = jnp.dot(qf, whigh, preferred_element_type=jnp.float32)
    out_ref[...] = low.astype(jnp.int32) | (high.astype(jnp.int32) << 16)


def _repack(q2):
    R = q2.shape[0]
    return pl.pallas_call(
        _repack_kernel,
        out_shape=jax.ShapeDtypeStruct((R, 128), jnp.int32),
        grid=(R // BRT,),
        in_specs=[pl.BlockSpec((BRT, 512), lambda i: (i, 0))],
        out_specs=pl.BlockSpec((BRT, 128), lambda i: (i, 0)),
    )(q2)


def _dequant_chunk(rows_v, amax_v, out_v, iota16):
    inv127 = jnp.float32(1.0 / 127.0)

    def grp(g, _):
        r0 = g * L
        scale16 = amax_v[pl.ds(r0, L)] * inv127
        for rr in range(L):
            r = r0 + rr
            words = rows_v[r]
            scale_b = jnp.broadcast_to(scale16[rr], (L,))
            obase = r * 64
            for j in range(4):
                if j < 3:
                    b = (words << (24 - 8 * j)) >> 24
                else:
                    b = words >> 24
                f = b.astype(jnp.float32) * scale_b
                plsc.store_scatter(out_v, [obase + 4 * iota16 + j], f)
        return 0

    lax.fori_loop(0, CHUNK // L, grp, 0)


def _make_sc_kernel(N, V):
    per_w = N // NW
    nchunks = per_w // CHUNK
    mesh = plsc.VectorSubcoreMesh(
        core_axis_name="c", subcore_axis_name="s", num_cores=NC, num_subcores=NS)

    @functools.partial(
        pl.kernel,
        out_type=jax.ShapeDtypeStruct((N * 64,), jnp.float32),
        mesh=mesh,
        compiler_params=pltpu.CompilerParams(
            use_tc_tiling_on_sc=False, needs_layout_passes=False),
        scratch_types=[
            pltpu.VMEM((CHUNK,), jnp.int32),
            pltpu.VMEM((CHUNK, 16), jnp.int32),
            pltpu.VMEM((CHUNK,), jnp.float32),
            pltpu.VMEM((CHUNK * 64,), jnp.float32),
            pltpu.SemaphoreType.DMA,
            pltpu.SemaphoreType.DMA,
        ],
    )
    def k(idx_hbm, wtab_hbm, amax_hbm, out_hbm,
          idx_v, rows_v, amax_v, out_v, sem_r, sem_a):
        wid = lax.axis_index("s") * NC + lax.axis_index("c")
        iota16 = lax.iota(jnp.int32, L)

        def body(ci, _):
            base = wid * per_w + ci * CHUNK
            pltpu.sync_copy(idx_hbm.at[pl.ds(base, CHUNK)], idx_v)
            for s in range(NSUB):
                pltpu.async_copy(
                    wtab_hbm.at[idx_v.at[pl.ds(s * IDXW, IDXW)]],
                    rows_v.at[pl.ds(s * IDXW, IDXW)], sem_r)
                pltpu.async_copy(
                    amax_hbm.at[idx_v.at[pl.ds(s * IDXW, IDXW)]],
                    amax_v.at[pl.ds(s * IDXW, IDXW)], sem_a)
            for s in range(NSUB):
                pltpu.make_async_copy(
                    wtab_hbm.at[idx_v.at[pl.ds(s * IDXW, IDXW)]],
                    rows_v.at[pl.ds(s * IDXW, IDXW)], sem_r).wait()
                pltpu.make_async_copy(
                    amax_hbm.at[idx_v.at[pl.ds(s * IDXW, IDXW)]],
                    amax_v.at[pl.ds(s * IDXW, IDXW)], sem_a).wait()
            _dequant_chunk(rows_v, amax_v, out_v, iota16)
            pltpu.sync_copy(out_v, out_hbm.at[pl.ds(base * 64, CHUNK * 64)])
            return 0

        lax.fori_loop(0, nchunks, body, 0)

    return k


def kernel(x, q_weight, absmax):
    B, S = x.shape
    V, D = q_weight.shape
    N = B * S
    idx = x.reshape(N).astype(jnp.int32)
    wtab = _repack(q_weight.reshape(V // 8, 8 * D)).reshape(V, 16)
    out = _make_sc_kernel(N, V)(idx, wtab, absmax)
    return out.reshape(B, S, D)


# speedup vs baseline: 8.5696x; 1.0086x over previous
import functools

import jax
import jax.numpy as jnp
from jax import lax
from jax.experimental import pallas as pl
from jax.experimental.pallas import tpu as pltpu
from jax.experimental.pallas import tpu_sc as plsc

NC = 2
NS = 16
NW = NC * NS
L = 16

IDXW = 128
CHUNK = 1280
NSUB = CHUNK // IDXW

BRT = 1000


def _repack_kernel(q_ref, out_ref):
    qf = (q_ref[...].astype(jnp.int32) & 255).astype(jnp.float32)
    c = jax.lax.broadcasted_iota(jnp.int32, (512, 128), 0)
    m4 = jax.lax.broadcasted_iota(jnp.int32, (512, 128), 1) * 4
    wlow = ((c == m4) + (c == m4 + 1) * 256).astype(jnp.float32)
    whigh = ((c == m4 + 2) + (c == m4 + 3) * 256).astype(jnp.float32)
    low = jnp.dot(qf, wlow, preferred_element_type=jnp.float32)
    high = jnp.dot(qf, whigh, preferred_element_type=jnp.float32)
    out_ref[...] = low.astype(jnp.int32) | (high.astype(jnp.int32) << 16)


def _repack(q2):
    R = q2.shape[0]
    return pl.pallas_call(
        _repack_kernel,
        out_shape=jax.ShapeDtypeStruct((R, 128), jnp.int32),
        grid=(R // BRT,),
        in_specs=[pl.BlockSpec((BRT, 512), lambda i: (i, 0))],
        out_specs=pl.BlockSpec((BRT, 128), lambda i: (i, 0)),
    )(q2)


def _dequant_chunk(rows_v, amax_v, out_v, iota16):
    inv127 = jnp.float32(1.0 / 127.0)

    def grp(g, _):
        r0 = g * L
        scale16 = amax_v[pl.ds(r0, L)] * inv127
        for rr in range(L):
            r = r0 + rr
            words = rows_v[r]
            scale_b = jnp.broadcast_to(scale16[rr], (L,))
            obase = r * 64
            for j in range(4):
                if j < 3:
                    b = (words << (24 - 8 * j)) >> 24
                else:
                    b = words >> 24
                f = b.astype(jnp.float32) * scale_b
                plsc.store_scatter(out_v, [obase + 4 * iota16 + j], f)
        return 0

    lax.fori_loop(0, CHUNK // L, grp, 0)


def _make_sc_kernel(N, V):
    per_w = N // NW
    nchunks = per_w // CHUNK
    mesh = plsc.VectorSubcoreMesh(
        core_axis_name="c", subcore_axis_name="s", num_cores=NC, num_subcores=NS)

    @functools.partial(
        pl.kernel,
        out_type=jax.ShapeDtypeStruct((N * 64,), jnp.float32),
        mesh=mesh,
        compiler_params=pltpu.CompilerParams(
            use_tc_tiling_on_sc=False, needs_layout_passes=False),
        scratch_types=[
            pltpu.VMEM((CHUNK,), jnp.int32),
            pltpu.VMEM((CHUNK, 16), jnp.int32),
            pltpu.VMEM((CHUNK,), jnp.float32),
            pltpu.VMEM((CHUNK * 64,), jnp.float32),
            pltpu.SemaphoreType.DMA,
            pltpu.SemaphoreType.DMA,
        ],
    )
    def k(idx_hbm, wtab_hbm, amax_hbm, out_hbm,
          idx_v, rows_v, amax_v, out_v, sem_r, sem_a):
        wid = lax.axis_index("s") * NC + lax.axis_index("c")
        iota16 = lax.iota(jnp.int32, L)

        def body(ci, _):
            base = wid * per_w + ci * CHUNK
            pltpu.sync_copy(idx_hbm.at[pl.ds(base, CHUNK)], idx_v)
            for s in range(NSUB):
                pltpu.async_copy(
                    wtab_hbm.at[idx_v.at[pl.ds(s * IDXW, IDXW)]],
                    rows_v.at[pl.ds(s * IDXW, IDXW)], sem_r)
                pltpu.async_copy(
                    amax_hbm.at[idx_v.at[pl.ds(s * IDXW, IDXW)]],
                    amax_v.at[pl.ds(s * IDXW, IDXW)], sem_a)
            for s in range(NSUB):
                pltpu.make_async_copy(
                    wtab_hbm.at[idx_v.at[pl.ds(s * IDXW, IDXW)]],
                    rows_v.at[pl.ds(s * IDXW, IDXW)], sem_r).wait()
                pltpu.make_async_copy(
                    amax_hbm.at[idx_v.at[pl.ds(s * IDXW, IDXW)]],
                    amax_v.at[pl.ds(s * IDXW, IDXW)], sem_a).wait()
            _dequant_chunk(rows_v, amax_v, out_v, iota16)
            pltpu.sync_copy(out_v, out_hbm.at[pl.ds(base * 64, CHUNK * 64)])
            return 0

        lax.fori_loop(0, nchunks, body, 0)

    return k


def kernel(x, q_weight, absmax):
    B, S = x.shape
    V, D = q_weight.shape
    N = B * S
    idx = x.reshape(N).astype(jnp.int32)
    wtab = _repack(q_weight.reshape(V // 8, 8 * D)).reshape(V, 16)
    out = _make_sc_kernel(N, V)(idx, wtab, absmax)
    return out.reshape(B, S, D)
